# baseline (speedup 1.0000x reference)
import functools

import jax
import jax.numpy as jnp
from jax import lax
from jax.experimental import pallas as pl
from jax.experimental.pallas import tpu as pltpu
from jax.experimental.pallas import tpu_sc as plsc

N_ROWS = 4096
N_COLS = 4096
NC = 2
NS = 16
NW = NC * NS
ROWS_PER_W = N_ROWS // NW
R = 8
CHUNKS = ROWS_PER_W // R
LANES = 16
G = N_COLS // LANES


@functools.partial(
    pl.kernel,
    mesh=plsc.VectorSubcoreMesh(core_axis_name="c", subcore_axis_name="s"),
    out_type=jax.ShapeDtypeStruct((N_ROWS * N_COLS,), jnp.float32),
    compiler_params=pltpu.CompilerParams(needs_layout_passes=False),
    scratch_types=[
        pltpu.VMEM((N_COLS,), jnp.int32),
        pltpu.VMEM((R * N_COLS,), jnp.float32),
        pltpu.VMEM((R * N_COLS,), jnp.float32),
    ],
)
def _permute(x_hbm, perm_hbm, out_hbm, perm_v, in_v, out_v):
    wid = lax.axis_index("s") * NC + lax.axis_index("c")
    pltpu.sync_copy(perm_hbm, perm_v)
    elem0 = wid * (ROWS_PER_W * N_COLS)

    def chunk_body(c, carry):
        base = elem0 + c * (R * N_COLS)
        pltpu.sync_copy(x_hbm.at[pl.ds(base, R * N_COLS)], in_v)

        def g_body(g, carry2):
            idx = perm_v[pl.ds(g * LANES, LANES)]
            for r in range(R):
                vals = plsc.load_gather(in_v, [idx + (r * N_COLS)])
                out_v[pl.ds(r * N_COLS + g * LANES, LANES)] = vals
            return carry2

        lax.fori_loop(0, G, g_body, 0)
        pltpu.sync_copy(out_v, out_hbm.at[pl.ds(base, R * N_COLS)])
        return carry

    lax.fori_loop(0, CHUNKS, chunk_body, 0)


def kernel(x, permutation):
    out = _permute(x.reshape(-1), permutation)
    return out.reshape(N_ROWS, N_COLS)


# speedup vs baseline: 1.4699x; 1.4699x over previous
import functools

import jax
import jax.numpy as jnp
from jax import lax
from jax.experimental import pallas as pl
from jax.experimental.pallas import tpu as pltpu
from jax.experimental.pallas import tpu_sc as plsc

N_ROWS = 4096
N_COLS = 4096
NC = 2
NS = 16
NW = NC * NS
ROWS_PER_W = N_ROWS // NW
R = 8
CHUNKS = ROWS_PER_W // R
LANES = 16
G = N_COLS // LANES


@functools.partial(
    pl.kernel,
    mesh=plsc.VectorSubcoreMesh(core_axis_name="c", subcore_axis_name="s"),
    out_type=jax.ShapeDtypeStruct((N_ROWS, N_COLS), jnp.float32),
    compiler_params=pltpu.CompilerParams(needs_layout_passes=False),
    scratch_types=[
        pltpu.VMEM((N_COLS,), jnp.int32),
        pltpu.VMEM((R, N_COLS), jnp.float32),
        pltpu.VMEM((R, N_COLS), jnp.float32),
    ],
)
def _permute(x_hbm, perm_hbm, out_hbm, perm_v, in_v, out_v):
    wid = lax.axis_index("s") * NC + lax.axis_index("c")
    pltpu.sync_copy(perm_hbm, perm_v)
    row0 = wid * ROWS_PER_W

    def chunk_body(c, carry):
        base = row0 + c * R
        pltpu.sync_copy(x_hbm.at[pl.ds(base, R)], in_v)

        def g_body(g, carry2):
            idx = perm_v[pl.ds(g * LANES, LANES)]
            for r in range(R):
                row_idx = jnp.full((LANES,), r, jnp.int32)
                vals = plsc.load_gather(in_v, [row_idx, idx])
                out_v[r, pl.ds(g * LANES, LANES)] = vals
            return carry2

        lax.fori_loop(0, G, g_body, 0)
        pltpu.sync_copy(out_v, out_hbm.at[pl.ds(base, R)])
        return carry

    lax.fori_loop(0, CHUNKS, chunk_body, 0)


def kernel(x, permutation):
    return _permute(x, permutation)


# speedup vs baseline: 1.6105x; 1.0957x over previous
import functools

import jax
import jax.numpy as jnp
from jax import lax
from jax.experimental import pallas as pl
from jax.experimental.pallas import tpu as pltpu
from jax.experimental.pallas import tpu_sc as plsc

N_ROWS = 4096
N_COLS = 4096
NC = 2
NS = 16
NW = NC * NS
ROWS_PER_W = N_ROWS // NW
R = 4
NCHUNK = ROWS_PER_W // R
LANES = 16
G = N_COLS // LANES


@functools.partial(
    pl.kernel,
    mesh=plsc.VectorSubcoreMesh(core_axis_name="c", subcore_axis_name="s"),
    out_type=jax.ShapeDtypeStruct((N_ROWS, N_COLS), jnp.float32),
    compiler_params=pltpu.CompilerParams(needs_layout_passes=False),
    scratch_types=[
        pltpu.VMEM((N_COLS,), jnp.int32),
        pltpu.VMEM((R, N_COLS), jnp.float32),
        pltpu.VMEM((R, N_COLS), jnp.float32),
        pltpu.VMEM((R, N_COLS), jnp.float32),
        pltpu.VMEM((R, N_COLS), jnp.float32),
        pltpu.SemaphoreType.DMA,
        pltpu.SemaphoreType.DMA,
        pltpu.SemaphoreType.DMA,
        pltpu.SemaphoreType.DMA,
    ],
)
def _permute(x_hbm, perm_hbm, out_hbm, perm_v,
             in0, in1, out0, out1, isem0, isem1, osem0, osem1):
    wid = lax.axis_index("s") * NC + lax.axis_index("c")
    pltpu.sync_copy(perm_hbm, perm_v)
    row0 = wid * ROWS_PER_W

    in_bufs = (in0, in1)
    out_bufs = (out0, out1)
    in_sems = (isem0, isem1)
    out_sems = (osem0, osem1)

    def in_src(c):
        return x_hbm.at[pl.ds(row0 + c * R, R)]

    def out_dst(c):
        return out_hbm.at[pl.ds(row0 + c * R, R)]

    def gather_chunk(src_v, dst_v):
        @pl.loop(0, G, unroll=4)
        def _(g):
            idx = perm_v[pl.ds(g * LANES, LANES)]
            for r in range(R):
                row_idx = jnp.full((LANES,), r, jnp.int32)
                vals = plsc.load_gather(src_v, [row_idx, idx])
                dst_v[r, pl.ds(g * LANES, LANES)] = vals

    pltpu.async_copy(in_src(0), in0, isem0)
    pltpu.async_copy(in_src(1), in1, isem1)
    for b in range(2):
        pltpu.make_async_copy(in_src(b), in_bufs[b], in_sems[b]).wait()
        gather_chunk(in_bufs[b], out_bufs[b])
        pltpu.async_copy(out_bufs[b], out_dst(b), out_sems[b])
        pltpu.async_copy(in_src(b + 2), in_bufs[b], in_sems[b])

    @pl.loop(0, (NCHUNK - 2) // 2)
    def _(cc):
        for b in range(2):
            c = 2 + cc * 2 + b
            pltpu.make_async_copy(in_src(c), in_bufs[b], in_sems[b]).wait()
            pltpu.make_async_copy(out_bufs[b], out_dst(c - 2),
                                  out_sems[b]).wait()
            gather_chunk(in_bufs[b], out_bufs[b])
            pltpu.async_copy(out_bufs[b], out_dst(c), out_sems[b])

            @pl.when(c + 2 < NCHUNK)
            def _():
                pltpu.async_copy(in_src(c + 2), in_bufs[b], in_sems[b])

    for b in range(2):
        pltpu.make_async_copy(out_bufs[b], out_dst(NCHUNK - 2 + b),
                              out_sems[b]).wait()


def kernel(x, permutation):
    return _permute(x, permutation)


# speedup vs baseline: 1.7453x; 1.0837x over previous
import functools

import jax
import jax.numpy as jnp
from jax import lax
from jax.experimental import pallas as pl
from jax.experimental.pallas import tpu as pltpu
from jax.experimental.pallas import tpu_sc as plsc

N_ROWS = 4096
N_COLS = 4096
NC = 2
NS = 16
NW = NC * NS
ROWS_PER_W = N_ROWS // NW
R = 4
NCHUNK = ROWS_PER_W // R
LANES = 16
G = N_COLS // LANES


@functools.partial(
    pl.kernel,
    mesh=plsc.VectorSubcoreMesh(core_axis_name="c", subcore_axis_name="s"),
    out_type=jax.ShapeDtypeStruct((N_ROWS, N_COLS), jnp.float32),
    compiler_params=pltpu.CompilerParams(needs_layout_passes=False),
    scratch_types=[
        pltpu.VMEM((N_COLS,), jnp.int32),
        pltpu.VMEM((R * N_COLS,), jnp.float32),
        pltpu.VMEM((R * N_COLS,), jnp.float32),
        pltpu.VMEM((R * N_COLS,), jnp.float32),
        pltpu.VMEM((R * N_COLS,), jnp.float32),
        pltpu.SemaphoreType.DMA,
        pltpu.SemaphoreType.DMA,
        pltpu.SemaphoreType.DMA,
        pltpu.SemaphoreType.DMA,
    ],
)
def _permute(x_hbm, perm_hbm, out_hbm, perm_v,
             in0, in1, out0, out1, isem0, isem1, osem0, osem1):
    wid = lax.axis_index("s") * NC + lax.axis_index("c")
    pltpu.sync_copy(perm_hbm, perm_v)
    row0 = wid * ROWS_PER_W

    in_bufs = (in0, in1)
    out_bufs = (out0, out1)
    in_sems = (isem0, isem1)
    out_sems = (osem0, osem1)

    def start_in(c, b):
        for r in range(R):
            pltpu.async_copy(x_hbm.at[row0 + c * R + r],
                             in_bufs[b].at[pl.ds(r * N_COLS, N_COLS)],
                             in_sems[b])

    def wait_in(c, b):
        for r in range(R):
            pltpu.make_async_copy(x_hbm.at[row0 + c * R + r],
                                  in_bufs[b].at[pl.ds(r * N_COLS, N_COLS)],
                                  in_sems[b]).wait()

    def start_out(c, b):
        for r in range(R):
            pltpu.async_copy(out_bufs[b].at[pl.ds(r * N_COLS, N_COLS)],
                             out_hbm.at[row0 + c * R + r],
                             out_sems[b])

    def wait_out(c, b):
        for r in range(R):
            pltpu.make_async_copy(out_bufs[b].at[pl.ds(r * N_COLS, N_COLS)],
                                  out_hbm.at[row0 + c * R + r],
                                  out_sems[b]).wait()

    def gather_chunk(src_v, dst_v):
        @pl.loop(0, G, unroll=8)
        def _(g):
            idx = perm_v[pl.ds(g * LANES, LANES)]
            for r in range(R):
                vals = plsc.load_gather(src_v, [idx + (r * N_COLS)])
                dst_v[pl.ds(r * N_COLS + g * LANES, LANES)] = vals

    start_in(0, 0)
    start_in(1, 1)
    for b in range(2):
        wait_in(b, b)
        gather_chunk(in_bufs[b], out_bufs[b])
        start_out(b, b)
        start_in(b + 2, b)

    @pl.loop(0, (NCHUNK - 2) // 2)
    def _(cc):
        for b in range(2):
            c = 2 + cc * 2 + b
            wait_in(c, b)
            wait_out(c - 2, b)
            gather_chunk(in_bufs[b], out_bufs[b])
            start_out(c, b)

            @pl.when(c + 2 < NCHUNK)
            def _():
                start_in(c + 2, b)

    for b in range(2):
        wait_out(NCHUNK - 2 + b, b)


def kernel(x, permutation):
    return _permute(x, permutation)


# speedup vs baseline: 4.3166x; 2.4733x over previous
import functools

import jax
import jax.numpy as jnp
from jax import lax
from jax.experimental import pallas as pl
from jax.experimental.pallas import tpu as pltpu
from jax.experimental.pallas import tpu_sc as plsc

N_ROWS = 4096
N_COLS = 4096
NC = 2
NS = 16
NW = NC * NS
ROWS_PER_W = N_ROWS // NW
R = 4
NCHUNK = ROWS_PER_W // R
LANES = 16
G = N_COLS // LANES


@functools.partial(
    pl.kernel,
    mesh=plsc.VectorSubcoreMesh(core_axis_name="c", subcore_axis_name="s"),
    out_type=jax.ShapeDtypeStruct((N_ROWS, N_COLS), jnp.float32),
    compiler_params=pltpu.CompilerParams(needs_layout_passes=False),
    scratch_types=[
        pltpu.VMEM((N_COLS,), jnp.int32),
        pltpu.VMEM((R * N_COLS,), jnp.float32),
        pltpu.VMEM((R * N_COLS,), jnp.float32),
        pltpu.VMEM((R * N_COLS,), jnp.float32),
        pltpu.VMEM((R * N_COLS,), jnp.float32),
        pltpu.SemaphoreType.DMA,
        pltpu.SemaphoreType.DMA,
        pltpu.SemaphoreType.DMA,
        pltpu.SemaphoreType.DMA,
    ],
)
def _permute(x_hbm, perm_hbm, out_hbm, perm_v,
             in0, in1, out0, out1, isem0, isem1, osem0, osem1):
    wid = lax.axis_index("s") * NC + lax.axis_index("c")
    pltpu.sync_copy(perm_hbm, perm_v)
    row0 = wid * ROWS_PER_W

    in_bufs = (in0, in1)
    out_bufs = (out0, out1)
    in_sems = (isem0, isem1)
    out_sems = (osem0, osem1)

    def start_in(c, b):
        for r in range(R):
            pltpu.async_copy(x_hbm.at[row0 + c * R + r],
                             in_bufs[b].at[pl.ds(r * N_COLS, N_COLS)],
                             in_sems[b])

    def wait_in(c, b):
        for r in range(R):
            pltpu.make_async_copy(x_hbm.at[row0 + c * R + r],
                                  in_bufs[b].at[pl.ds(r * N_COLS, N_COLS)],
                                  in_sems[b]).wait()

    def start_out(c, b):
        for r in range(R):
            pltpu.async_copy(out_bufs[b].at[pl.ds(r * N_COLS, N_COLS)],
                             out_hbm.at[row0 + c * R + r],
                             out_sems[b])

    def wait_out(c, b):
        for r in range(R):
            pltpu.make_async_copy(out_bufs[b].at[pl.ds(r * N_COLS, N_COLS)],
                                  out_hbm.at[row0 + c * R + r],
                                  out_sems[b]).wait()

    U = 4

    def gather_chunk(src_v, dst_v):
        @pl.loop(0, G // U)
        def _(gu):
            g0 = gu * U
            idxs = [perm_v[pl.ds((g0 + u) * LANES, LANES)] for u in range(U)]
            gidxs = [[idxs[u] + (r * N_COLS) if r else idxs[u]
                      for r in range(R)] for u in range(U)]
            vals = [[plsc.load_gather(src_v, [gidxs[u][r]])
                     for r in range(R)] for u in range(U)]
            for u in range(U):
                for r in range(R):
                    dst_v[pl.ds(r * N_COLS + (g0 + u) * LANES, LANES)] = \
                        vals[u][r]

    start_in(0, 0)
    start_in(1, 1)
    for b in range(2):
        wait_in(b, b)
        gather_chunk(in_bufs[b], out_bufs[b])
        start_out(b, b)
        start_in(b + 2, b)

    @pl.loop(0, (NCHUNK - 2) // 2)
    def _(cc):
        for b in range(2):
            c = 2 + cc * 2 + b
            wait_in(c, b)
            wait_out(c - 2, b)
            gather_chunk(in_bufs[b], out_bufs[b])
            start_out(c, b)

            @pl.when(c + 2 < NCHUNK)
            def _():
                start_in(c + 2, b)

    for b in range(2):
        wait_out(NCHUNK - 2 + b, b)


def kernel(x, permutation):
    return _permute(x, permutation)


# speedup vs baseline: 4.8533x; 1.1243x over previous
import functools

import jax
import jax.numpy as jnp
from jax import lax
from jax.experimental import pallas as pl
from jax.experimental.pallas import tpu as pltpu
from jax.experimental.pallas import tpu_sc as plsc

N_ROWS = 4096
N_COLS = 4096
NC = 2
NS = 16
NW = NC * NS
ROWS_PER_W = N_ROWS // NW
R = 4
NCHUNK = ROWS_PER_W // R
LANES = 16
G = N_COLS // LANES


@functools.partial(
    pl.kernel,
    mesh=plsc.VectorSubcoreMesh(core_axis_name="c", subcore_axis_name="s"),
    out_type=jax.ShapeDtypeStruct((N_ROWS, N_COLS), jnp.float32),
    compiler_params=pltpu.CompilerParams(needs_layout_passes=False),
    scratch_types=[
        pltpu.VMEM((N_COLS,), jnp.int32),
        pltpu.VMEM((R * N_COLS,), jnp.float32),
        pltpu.VMEM((R * N_COLS,), jnp.float32),
        pltpu.VMEM((R * N_COLS,), jnp.float32),
        pltpu.VMEM((R * N_COLS,), jnp.float32),
        pltpu.SemaphoreType.DMA,
        pltpu.SemaphoreType.DMA,
        pltpu.SemaphoreType.DMA,
        pltpu.SemaphoreType.DMA,
    ],
)
def _permute(x_hbm, perm_hbm, out_hbm, perm_v,
             in0, in1, out0, out1, isem0, isem1, osem0, osem1):
    wid = lax.axis_index("s") * NC + lax.axis_index("c")
    pltpu.sync_copy(perm_hbm, perm_v)
    row0 = wid * ROWS_PER_W

    in_bufs = (in0, in1)
    out_bufs = (out0, out1)
    in_sems = (isem0, isem1)
    out_sems = (osem0, osem1)

    def start_in(c, b):
        for r in range(R):
            pltpu.async_copy(x_hbm.at[row0 + c * R + r],
                             in_bufs[b].at[pl.ds(r * N_COLS, N_COLS)],
                             in_sems[b])

    def wait_in(c, b):
        for r in range(R):
            pltpu.make_async_copy(x_hbm.at[row0 + c * R + r],
                                  in_bufs[b].at[pl.ds(r * N_COLS, N_COLS)],
                                  in_sems[b]).wait()

    def start_out(c, b):
        for r in range(R):
            pltpu.async_copy(out_bufs[b].at[pl.ds(r * N_COLS, N_COLS)],
                             out_hbm.at[row0 + c * R + r],
                             out_sems[b])

    def wait_out(c, b):
        for r in range(R):
            pltpu.make_async_copy(out_bufs[b].at[pl.ds(r * N_COLS, N_COLS)],
                                  out_hbm.at[row0 + c * R + r],
                                  out_sems[b]).wait()

    U = 4

    def gather_chunk(src_v, dst_v):
        @plsc.parallel_loop(0, G // U)
        def _(gu):
            g0 = gu * U
            idxs = [perm_v[pl.ds((g0 + u) * LANES, LANES)] for u in range(U)]
            gidxs = [[idxs[u] + (r * N_COLS) if r else idxs[u]
                      for r in range(R)] for u in range(U)]
            vals = [[plsc.load_gather(src_v, [gidxs[u][r]])
                     for r in range(R)] for u in range(U)]
            for u in range(U):
                for r in range(R):
                    dst_v[pl.ds(r * N_COLS + (g0 + u) * LANES, LANES)] = \
                        vals[u][r]

    start_in(0, 0)
    start_in(1, 1)
    for b in range(2):
        wait_in(b, b)
        gather_chunk(in_bufs[b], out_bufs[b])
        start_out(b, b)
        start_in(b + 2, b)

    @pl.loop(0, (NCHUNK - 2) // 2)
    def _(cc):
        for b in range(2):
            c = 2 + cc * 2 + b
            wait_in(c, b)
            wait_out(c - 2, b)
            gather_chunk(in_bufs[b], out_bufs[b])
            start_out(c, b)

            @pl.when(c + 2 < NCHUNK)
            def _():
                start_in(c + 2, b)

    for b in range(2):
        wait_out(NCHUNK - 2 + b, b)


def kernel(x, permutation):
    return _permute(x, permutation)


# speedup vs baseline: 4.9741x; 1.0249x over previous
import functools

import jax
import jax.numpy as jnp
from jax import lax
from jax.experimental import pallas as pl
from jax.experimental.pallas import tpu as pltpu
from jax.experimental.pallas import tpu_sc as plsc

N_ROWS = 4096
N_COLS = 4096
NC = 2
NS = 16
NW = NC * NS
ROWS_PER_W = N_ROWS // NW
R = 8
NCHUNK = ROWS_PER_W // R
LANES = 16
HCOLS = N_COLS // 2
HG = HCOLS // LANES
U = 2


@functools.partial(
    pl.kernel,
    mesh=plsc.VectorSubcoreMesh(core_axis_name="c", subcore_axis_name="s"),
    out_type=jax.ShapeDtypeStruct((N_ROWS, N_COLS), jnp.float32),
    compiler_params=pltpu.CompilerParams(needs_layout_passes=False),
    scratch_types=[
        pltpu.VMEM((N_COLS,), jnp.int32),
        pltpu.VMEM((R, N_COLS), jnp.float32),
        pltpu.VMEM((R, N_COLS), jnp.float32),
        pltpu.VMEM((R, HCOLS), jnp.float32),
        pltpu.VMEM((R, HCOLS), jnp.float32),
        pltpu.SemaphoreType.DMA,
        pltpu.SemaphoreType.DMA,
        pltpu.SemaphoreType.DMA,
        pltpu.SemaphoreType.DMA,
    ],
)
def _permute(x_hbm, perm_hbm, out_hbm, perm_v,
             in0, in1, outh0, outh1, isem0, isem1, osem0, osem1):
    wid = lax.axis_index("s") * NC + lax.axis_index("c")
    pltpu.sync_copy(perm_hbm, perm_v)
    row0 = wid * ROWS_PER_W

    in_bufs = (in0, in1)
    in_sems = (isem0, isem1)
    out_bufs = (outh0, outh1)
    out_sems = (osem0, osem1)
    row_ids = [jnp.full((LANES,), r, jnp.int32) for r in range(R)]

    def start_in(c, b):
        pltpu.async_copy(x_hbm.at[pl.ds(row0 + c * R, R)], in_bufs[b],
                         in_sems[b])

    def wait_in(c, b):
        pltpu.make_async_copy(x_hbm.at[pl.ds(row0 + c * R, R)], in_bufs[b],
                              in_sems[b]).wait()

    def out_dst(c, h):
        return out_hbm.at[pl.ds(row0 + c * R, R), pl.ds(h * HCOLS, HCOLS)]

    def start_out(c, h):
        pltpu.async_copy(out_bufs[h], out_dst(c, h), out_sems[h])

    def wait_out(c, h):
        pltpu.make_async_copy(out_bufs[h], out_dst(c, h), out_sems[h]).wait()

    def gather_half(src_v, h):
        dst_v = out_bufs[h]

        @plsc.parallel_loop(0, HG // U)
        def _(gu):
            g0 = gu * U
            idxs = [perm_v[pl.ds((h * HG + g0 + u) * LANES, LANES)]
                    for u in range(U)]
            vals = [[plsc.load_gather(src_v, [row_ids[r], idxs[u]])
                     for r in range(R)] for u in range(U)]
            for u in range(U):
                for r in range(R):
                    dst_v[r, pl.ds((g0 + u) * LANES, LANES)] = vals[u][r]

    start_in(0, 0)
    start_in(1, 1)
    wait_in(0, 0)
    for h in range(2):
        gather_half(in_bufs[0], h)
        start_out(0, h)
    start_in(2, 0)

    @pl.loop(0, (NCHUNK - 2) // 2)
    def _(cc):
        for b in range(2):
            c = 1 + cc * 2 + b
            bb = (1 + b) % 2
            wait_in(c, bb)
            for h in range(2):
                wait_out(c - 1, h)
                gather_half(in_bufs[bb], h)
                start_out(c, h)

            @pl.when(c + 2 < NCHUNK)
            def _():
                start_in(c + 2, bb)

    c = NCHUNK - 1
    wait_in(c, 1)
    for h in range(2):
        wait_out(c - 1, h)
        gather_half(in_bufs[1], h)
        start_out(c, h)
    for h in range(2):
        wait_out(c, h)


def kernel(x, permutation):
    return _permute(x, permutation)
